# baseline (speedup 1.0000x reference)
import functools

import jax
import jax.numpy as jnp
from jax import lax
from jax.experimental import pallas as pl
from jax.experimental.pallas import tpu as pltpu, tpu_sc as plsc

N = 10000
NP = 10112
E = 320000
D = 128
D_EDGE = 16
K = 128
NCHUNK = E // K
NW = 32
ROWS_PER_SUB = NP // 16
EPT = E // NW
CPT = 79
E2 = 324000
MAX_CHUNKS = (NCHUNK + NW - 1) // NW


def _prep_body(x_ref, g_ref, b_ref, w_ref, xn_ref, m0_ref):
    xb = x_ref[...]
    mean = jnp.mean(xb, axis=0, keepdims=True)
    xc = xb - mean
    var = jnp.mean(xc * xc, axis=0, keepdims=True)
    xn = xc * lax.rsqrt(var + 1e-5) * g_ref[...] + b_ref[...]
    xn_ref[...] = xn
    m0_ref[...] = jnp.dot(xn, w_ref[...], preferred_element_type=jnp.float32)


def _tc_prep(x, g2, b2, w0):
    return pl.pallas_call(
        _prep_body,
        out_shape=[jax.ShapeDtypeStruct((N, D), jnp.float32)] * 2,
    )(x, g2, b2, w0)


_BE = 4000


def _ea_body(a_ref, w_ref, o_ref):
    o_ref[...] = lax.dot_general(
        a_ref[...], w_ref[...], (((1,), (1,)), ((), ())),
        preferred_element_type=jnp.float32)


def _tc_ea(edge_attr, w_edge):
    return pl.pallas_call(
        _ea_body,
        grid=(E2 // _BE,),
        in_specs=[
            pl.BlockSpec((_BE, D_EDGE), lambda i: (i, 0)),
            pl.BlockSpec((D, D_EDGE), lambda i: (0, 0)),
        ],
        out_specs=pl.BlockSpec((_BE, D), lambda i: (i, 0)),
        out_shape=jax.ShapeDtypeStruct((E2, D), jnp.float32),
    )(edge_attr, w_edge)


def _sc_body(m_hbm, ea_hbm, eip_hbm, zd_hbm, out_p,
             idx, rows, eab, agg_sh, sem_i, sem_g, sem_e, sem_s):
    c = lax.axis_index("c")
    s = lax.axis_index("s")
    wid = c * 16 + s
    row0 = pl.multiple_of(s * ROWS_PER_SUB, 8)

    pltpu.sync_copy(zd_hbm.at[pl.ds(row0, ROWS_PER_SUB)],
                    agg_sh.at[pl.ds(row0, ROWS_PER_SUB)])
    plsc.subcore_barrier()

    def issue_idx(g, i3):
        pltpu.async_copy(eip_hbm.at[wid, g], idx.at[i3], sem_i.at[i3])

    def wait_idx(i3):
        pltpu.make_async_copy(eip_hbm.at[0, 0], idx.at[i3],
                              sem_i.at[i3]).wait()

    def issue_gather(b, i3):
        pltpu.async_copy(m_hbm.at[idx.at[i3, 0]], rows.at[b], sem_g.at[b])

    def wait_gather(b, i3):
        pltpu.make_async_copy(m_hbm.at[idx.at[i3, 0]], rows.at[b],
                              sem_g.at[b]).wait()

    def issue_ea(g):
        base = pl.multiple_of(wid * EPT + g * K, 16)
        pltpu.async_copy(ea_hbm.at[pl.ds(base, K)], eab, sem_e)

    def issue_scatter(b, i3):
        pltpu.async_copy(rows.at[b], agg_sh.at[idx.at[i3, 1]], sem_s.at[b],
                         add=True)

    def wait_scatter(b, i3):
        pltpu.make_async_copy(rows.at[b], agg_sh.at[idx.at[i3, 1]],
                              sem_s.at[b]).wait()

    issue_idx(0, 0)
    wait_idx(0)
    issue_gather(0, 0)
    issue_ea(0)
    issue_idx(1, 1)

    def step(g, carry):
        b = lax.rem(g, 2)
        i3 = lax.rem(g, 3)

        @pl.when(g + 2 < CPT)
        def _():
            issue_idx(g + 2, lax.rem(g + 2, 3))

        wait_gather(b, i3)
        pltpu.make_async_copy(ea_hbm.at[pl.ds(0, K)], eab, sem_e).wait()

        @pl.when(g + 1 < CPT)
        def _():
            nb = lax.rem(g + 1, 2)
            ni3 = lax.rem(g + 1, 3)

            @pl.when(g >= 1)
            def _():
                wait_scatter(nb, lax.rem(g - 1, 3))

            wait_idx(ni3)
            issue_gather(nb, ni3)

        def erow(r, carry2):
            for j in range(D // 16):
                sl = pl.ds(j * 16, 16)
                v = rows[b, r, sl] + eab[r, sl]
                rows[b, r, sl] = jnp.maximum(v, 0.0)
            return carry2

        lax.fori_loop(0, K, erow, 0)
        issue_scatter(b, i3)

        @pl.when(g + 1 < CPT)
        def _():
            issue_ea(g + 1)

        return carry

    lax.fori_loop(0, CPT, step, 0)
    wait_scatter(0, lax.rem(CPT - 1, 3))
    wait_scatter(1, lax.rem(CPT - 2, 3))
    plsc.subcore_barrier()
    pltpu.sync_copy(agg_sh.at[pl.ds(row0, ROWS_PER_SUB)], out_p.at[c, s])


_sc_agg = functools.partial(
    pl.kernel,
    mesh=plsc.VectorSubcoreMesh(core_axis_name="c", subcore_axis_name="s"),
    out_type=jax.ShapeDtypeStruct((2, 16, ROWS_PER_SUB, D), jnp.float32),
    scratch_types=(
        [pltpu.VMEM((3, 2, K), jnp.int32)]
        + [pltpu.VMEM((2, K, D), jnp.float32)]
        + [pltpu.VMEM((K, D), jnp.float32)]
        + [pltpu.VMEM_SHARED((NP, D), jnp.float32)]
        + [pltpu.SemaphoreType.DMA((3,))]
        + [pltpu.SemaphoreType.DMA((2,))]
        + [pltpu.SemaphoreType.DMA]
        + [pltpu.SemaphoreType.DMA((2,))]
    ),
)(_sc_body)


def _sc_cnt_body(dst_hbm, zd_hbm, ones_hbm, out_c, idx_d, ones_v, cnt_sh):
    c = lax.axis_index("c")
    s = lax.axis_index("s")
    wid = c * 16 + s
    row0 = pl.multiple_of(s * ROWS_PER_SUB, 8)

    pltpu.sync_copy(zd_hbm.at[pl.ds(row0, ROWS_PER_SUB)],
                    cnt_sh.at[pl.ds(row0, ROWS_PER_SUB)])
    pltpu.sync_copy(ones_hbm, ones_v)
    plsc.subcore_barrier()

    def chunk(g, carry):
        ci = wid + NW * g

        @pl.when(ci < NCHUNK)
        def _():
            base = pl.multiple_of(ci * K, K)
            pltpu.sync_copy(dst_hbm.at[pl.ds(base, K)], idx_d)
            pltpu.sync_copy(ones_v, cnt_sh.at[idx_d], add=True)

        return carry

    lax.fori_loop(0, MAX_CHUNKS, chunk, 0)
    plsc.subcore_barrier()
    pltpu.sync_copy(cnt_sh.at[pl.ds(row0, ROWS_PER_SUB)], out_c.at[c, s])


_sc_cnt = functools.partial(
    pl.kernel,
    mesh=plsc.VectorSubcoreMesh(core_axis_name="c", subcore_axis_name="s"),
    out_type=jax.ShapeDtypeStruct((2, 16, ROWS_PER_SUB, D), jnp.float32),
    scratch_types=[
        pltpu.VMEM((K,), jnp.int32),
        pltpu.VMEM((K, D), jnp.float32),
        pltpu.VMEM_SHARED((NP, D), jnp.float32),
    ],
)(_sc_cnt_body)


_BN = 1000


def _gru_body(final, p_ref, c_ref, h_ref, wih_ref, whh_ref, bih_ref, bhh_ref,
              wg_ref, h_out, *m_out):
    p = p_ref[...]
    agg_sum = p[0] + p[1]
    cp = c_ref[...]
    cnt = cp[0, :, 0:1] + cp[1, :, 0:1]
    denom = jnp.maximum(cnt, 1.0)
    agg = agg_sum / denom
    hb = h_ref[...]
    gi = lax.dot_general(agg, wih_ref[...], (((1,), (1,)), ((), ())),
                         preferred_element_type=jnp.float32) + bih_ref[...]
    gh = lax.dot_general(hb, whh_ref[...], (((1,), (1,)), ((), ())),
                         preferred_element_type=jnp.float32) + bhh_ref[...]
    r = jax.nn.sigmoid(gi[:, :D] + gh[:, :D])
    z = jax.nn.sigmoid(gi[:, D:2 * D] + gh[:, D:2 * D])
    n = jnp.tanh(gi[:, 2 * D:] + r * gh[:, 2 * D:])
    hn = (1.0 - z) * n + z * hb
    if final:
        h_out[...] = jnp.maximum(hn, 0.0)
    else:
        h_out[...] = hn
        m_out[0][...] = jnp.dot(hn, wg_ref[...],
                                preferred_element_type=jnp.float32)


def _make_gru(final):
    out_shape = [jax.ShapeDtypeStruct((N, D), jnp.float32)]
    if not final:
        out_shape.append(jax.ShapeDtypeStruct((N, D), jnp.float32))
    return pl.pallas_call(
        functools.partial(_gru_body, final),
        grid=(N // _BN,),
        in_specs=[
            pl.BlockSpec((2, _BN, D), lambda i: (0, i, 0)),
            pl.BlockSpec((2, _BN, D), lambda i: (0, i, 0)),
            pl.BlockSpec((_BN, D), lambda i: (i, 0)),
            pl.BlockSpec((3 * D, D), lambda i: (0, 0)),
            pl.BlockSpec((3 * D, D), lambda i: (0, 0)),
            pl.BlockSpec((1, 3 * D), lambda i: (0, 0)),
            pl.BlockSpec((1, 3 * D), lambda i: (0, 0)),
            pl.BlockSpec((D, D), lambda i: (0, 0)),
        ],
        out_specs=[pl.BlockSpec((_BN, D), lambda i: (i, 0))] * len(out_shape),
        out_shape=out_shape,
    )


def kernel(x, edge_index, edge_attr, bn_gamma, bn_beta, W_edge, ggc_weight,
           w_ih, w_hh, b_ih, b_hh):
    src = edge_index[0].astype(jnp.int32)
    dst = edge_index[1].astype(jnp.int32)
    srcp = jnp.pad(src.reshape(NW, EPT), ((0, 0), (0, CPT * K - EPT)),
                   constant_values=0).reshape(NW, CPT, K)
    dstp = jnp.pad(dst.reshape(NW, EPT), ((0, 0), (0, CPT * K - EPT)),
                   constant_values=N).reshape(NW, CPT, K)
    eip = jnp.stack([srcp, dstp], axis=2)
    ea_attr = jnp.pad(edge_attr, ((0, E2 - E), (0, 0)))
    g2 = bn_gamma.reshape(1, D)
    b2 = bn_beta.reshape(1, D)
    bih2 = b_ih.reshape(1, 3 * D)
    bhh2 = b_hh.reshape(1, 3 * D)
    zd = jnp.zeros((NP, D), jnp.float32)
    ones = jnp.ones((K, D), jnp.float32)

    x_norm, m0 = _tc_prep(x, g2, b2, ggc_weight[0])
    ea = _tc_ea(ea_attr, W_edge)

    c0 = _sc_cnt(dst, zd, ones)
    p0 = _sc_agg(m0, ea, eip, zd)
    p0 = p0.reshape(2, NP, D)
    c0 = c0.reshape(2, NP, D)
    h1, m1 = _make_gru(False)(p0, c0, x_norm, w_ih, w_hh, bih2, bhh2,
                              ggc_weight[1])

    p1 = _sc_agg(m1, ea, eip, zd)
    p1 = p1.reshape(2, NP, D)
    (out,) = _make_gru(True)(p1, c0, h1, w_ih, w_hh, bih2, bhh2,
                             ggc_weight[1])
    return out


# speedup vs baseline: 1.3799x; 1.3799x over previous
import functools

import jax
import jax.numpy as jnp
from jax import lax
from jax.experimental import pallas as pl
from jax.experimental.pallas import tpu as pltpu, tpu_sc as plsc

N = 10000
NP = 10112
E = 320000
D = 128
D_EDGE = 16
K = 128
NCHUNK = E // K
NW = 32
ROWS_PER_SUB = NP // 16
EPT = E // NW
CPT = 79
E2 = 324000
MAX_CHUNKS = (NCHUNK + NW - 1) // NW


def _prep_body(x_ref, g_ref, b_ref, w_ref, xn_ref, m0_ref):
    xb = x_ref[...]
    mean = jnp.mean(xb, axis=0, keepdims=True)
    xc = xb - mean
    var = jnp.mean(xc * xc, axis=0, keepdims=True)
    xn = xc * lax.rsqrt(var + 1e-5) * g_ref[...] + b_ref[...]
    xn_ref[...] = xn
    m0_ref[...] = jnp.dot(xn, w_ref[...], preferred_element_type=jnp.float32)


def _tc_prep(x, g2, b2, w0):
    return pl.pallas_call(
        _prep_body,
        out_shape=[jax.ShapeDtypeStruct((N, D), jnp.float32)] * 2,
    )(x, g2, b2, w0)


_BE = 4000


def _ea_body(a_ref, w_ref, o_ref):
    o_ref[...] = lax.dot_general(
        a_ref[...], w_ref[...], (((1,), (1,)), ((), ())),
        preferred_element_type=jnp.float32)


def _tc_ea(edge_attr, w_edge):
    return pl.pallas_call(
        _ea_body,
        grid=(E2 // _BE,),
        in_specs=[
            pl.BlockSpec((_BE, D_EDGE), lambda i: (i, 0)),
            pl.BlockSpec((D, D_EDGE), lambda i: (0, 0)),
        ],
        out_specs=pl.BlockSpec((_BE, D), lambda i: (i, 0)),
        out_shape=jax.ShapeDtypeStruct((E2, D), jnp.float32),
    )(edge_attr, w_edge)


def _sc_body(m_hbm, ea_hbm, eip_hbm, zd_hbm, out_p,
             idx, rows, eab, agg_sh, sem_i, sem_g, sem_e):
    c = lax.axis_index("c")
    s = lax.axis_index("s")
    wid = c * 16 + s
    row0 = pl.multiple_of(s * ROWS_PER_SUB, 8)

    pltpu.sync_copy(zd_hbm.at[pl.ds(row0, ROWS_PER_SUB)],
                    agg_sh.at[pl.ds(row0, ROWS_PER_SUB)])
    plsc.subcore_barrier()

    def step(g, carry):
        pltpu.async_copy(eip_hbm.at[wid, g], idx, sem_i).wait()
        h1 = pltpu.async_copy(m_hbm.at[idx.at[0]], rows, sem_g)
        base = pl.multiple_of(wid * EPT + g * K, 16)
        h2 = pltpu.async_copy(ea_hbm.at[pl.ds(base, K)], eab, sem_e)
        h1.wait()
        h2.wait()

        def erow(r, carry2):
            for j in range(D // 16):
                sl = pl.ds(j * 16, 16)
                v = rows[r, sl] + eab[r, sl]
                rows[r, sl] = jnp.maximum(v, 0.0)
            return carry2

        lax.fori_loop(0, K, erow, 0)
        pltpu.sync_copy(rows, agg_sh.at[idx.at[1]], add=True)
        return carry

    lax.fori_loop(0, CPT, step, 0)
    plsc.subcore_barrier()
    pltpu.sync_copy(agg_sh.at[pl.ds(row0, ROWS_PER_SUB)], out_p.at[c, s])


_sc_agg = functools.partial(
    pl.kernel,
    mesh=plsc.VectorSubcoreMesh(core_axis_name="c", subcore_axis_name="s"),
    out_type=jax.ShapeDtypeStruct((2, 16, ROWS_PER_SUB, D), jnp.float32),
    scratch_types=(
        [pltpu.VMEM((2, K), jnp.int32)]
        + [pltpu.VMEM((K, D), jnp.float32)] * 2
        + [pltpu.VMEM_SHARED((NP, D), jnp.float32)]
        + [pltpu.SemaphoreType.DMA] * 3
    ),
)(_sc_body)


def _sc_cnt_body(dst_hbm, zd_hbm, ones_hbm, out_c, idx_d, ones_v, cnt_sh):
    c = lax.axis_index("c")
    s = lax.axis_index("s")
    wid = c * 16 + s
    row0 = pl.multiple_of(s * ROWS_PER_SUB, 8)

    pltpu.sync_copy(zd_hbm.at[pl.ds(row0, ROWS_PER_SUB)],
                    cnt_sh.at[pl.ds(row0, ROWS_PER_SUB)])
    pltpu.sync_copy(ones_hbm, ones_v)
    plsc.subcore_barrier()

    def chunk(g, carry):
        ci = wid + NW * g

        @pl.when(ci < NCHUNK)
        def _():
            base = pl.multiple_of(ci * K, K)
            pltpu.sync_copy(dst_hbm.at[pl.ds(base, K)], idx_d)
            pltpu.sync_copy(ones_v, cnt_sh.at[idx_d], add=True)

        return carry

    lax.fori_loop(0, MAX_CHUNKS, chunk, 0)
    plsc.subcore_barrier()
    pltpu.sync_copy(cnt_sh.at[pl.ds(row0, ROWS_PER_SUB)], out_c.at[c, s])


_sc_cnt = functools.partial(
    pl.kernel,
    mesh=plsc.VectorSubcoreMesh(core_axis_name="c", subcore_axis_name="s"),
    out_type=jax.ShapeDtypeStruct((2, 16, ROWS_PER_SUB, D), jnp.float32),
    scratch_types=[
        pltpu.VMEM((K,), jnp.int32),
        pltpu.VMEM((K, D), jnp.float32),
        pltpu.VMEM_SHARED((NP, D), jnp.float32),
    ],
)(_sc_cnt_body)


_BN = 1000


def _gru_body(final, p_ref, c_ref, h_ref, wih_ref, whh_ref, bih_ref, bhh_ref,
              wg_ref, h_out, *m_out):
    p = p_ref[...]
    agg_sum = p[0] + p[1]
    cp = c_ref[...]
    cnt = cp[0, :, 0:1] + cp[1, :, 0:1]
    denom = jnp.maximum(cnt, 1.0)
    agg = agg_sum / denom
    hb = h_ref[...]
    gi = lax.dot_general(agg, wih_ref[...], (((1,), (1,)), ((), ())),
                         preferred_element_type=jnp.float32) + bih_ref[...]
    gh = lax.dot_general(hb, whh_ref[...], (((1,), (1,)), ((), ())),
                         preferred_element_type=jnp.float32) + bhh_ref[...]
    r = jax.nn.sigmoid(gi[:, :D] + gh[:, :D])
    z = jax.nn.sigmoid(gi[:, D:2 * D] + gh[:, D:2 * D])
    n = jnp.tanh(gi[:, 2 * D:] + r * gh[:, 2 * D:])
    hn = (1.0 - z) * n + z * hb
    if final:
        h_out[...] = jnp.maximum(hn, 0.0)
    else:
        h_out[...] = hn
        m_out[0][...] = jnp.dot(hn, wg_ref[...],
                                preferred_element_type=jnp.float32)


def _make_gru(final):
    out_shape = [jax.ShapeDtypeStruct((N, D), jnp.float32)]
    if not final:
        out_shape.append(jax.ShapeDtypeStruct((N, D), jnp.float32))
    return pl.pallas_call(
        functools.partial(_gru_body, final),
        grid=(N // _BN,),
        in_specs=[
            pl.BlockSpec((2, _BN, D), lambda i: (0, i, 0)),
            pl.BlockSpec((2, _BN, D), lambda i: (0, i, 0)),
            pl.BlockSpec((_BN, D), lambda i: (i, 0)),
            pl.BlockSpec((3 * D, D), lambda i: (0, 0)),
            pl.BlockSpec((3 * D, D), lambda i: (0, 0)),
            pl.BlockSpec((1, 3 * D), lambda i: (0, 0)),
            pl.BlockSpec((1, 3 * D), lambda i: (0, 0)),
            pl.BlockSpec((D, D), lambda i: (0, 0)),
        ],
        out_specs=[pl.BlockSpec((_BN, D), lambda i: (i, 0))] * len(out_shape),
        out_shape=out_shape,
    )


def kernel(x, edge_index, edge_attr, bn_gamma, bn_beta, W_edge, ggc_weight,
           w_ih, w_hh, b_ih, b_hh):
    src = edge_index[0].astype(jnp.int32)
    dst = edge_index[1].astype(jnp.int32)
    srcp = jnp.pad(src.reshape(NW, EPT), ((0, 0), (0, CPT * K - EPT)),
                   constant_values=0).reshape(NW, CPT, K)
    dstp = jnp.pad(dst.reshape(NW, EPT), ((0, 0), (0, CPT * K - EPT)),
                   constant_values=N).reshape(NW, CPT, K)
    eip = jnp.stack([srcp, dstp], axis=2)
    ea_attr = jnp.pad(edge_attr, ((0, E2 - E), (0, 0)))
    g2 = bn_gamma.reshape(1, D)
    b2 = bn_beta.reshape(1, D)
    bih2 = b_ih.reshape(1, 3 * D)
    bhh2 = b_hh.reshape(1, 3 * D)
    zd = jnp.zeros((NP, D), jnp.float32)
    ones = jnp.ones((K, D), jnp.float32)

    x_norm, m0 = _tc_prep(x, g2, b2, ggc_weight[0])
    ea = _tc_ea(ea_attr, W_edge)

    c0 = _sc_cnt(dst, zd, ones)
    p0 = _sc_agg(m0, ea, eip, zd)
    p0 = p0.reshape(2, NP, D)
    c0 = c0.reshape(2, NP, D)
    h1, m1 = _make_gru(False)(p0, c0, x_norm, w_ih, w_hh, bih2, bhh2,
                              ggc_weight[1])

    p1 = _sc_agg(m1, ea, eip, zd)
    p1 = p1.reshape(2, NP, D)
    (out,) = _make_gru(True)(p1, c0, h1, w_ih, w_hh, bih2, bhh2,
                             ggc_weight[1])
    return out


# speedup vs baseline: 1.4657x; 1.0622x over previous
import functools

import jax
import jax.numpy as jnp
from jax import lax
from jax.experimental import pallas as pl
from jax.experimental.pallas import tpu as pltpu, tpu_sc as plsc

N = 10000
NP = 10112
E = 320000
D = 128
D_EDGE = 16
K = 128
NCHUNK = E // K
NW = 32
ROWS_PER_SUB = NP // 16
EPT = E // NW
CPT = 79
E2 = 324000
MAX_CHUNKS = (NCHUNK + NW - 1) // NW


def _prep_body(x_ref, g_ref, b_ref, w_ref, xn_ref, m0_ref):
    xb = x_ref[...]
    mean = jnp.mean(xb, axis=0, keepdims=True)
    xc = xb - mean
    var = jnp.mean(xc * xc, axis=0, keepdims=True)
    xn = xc * lax.rsqrt(var + 1e-5) * g_ref[...] + b_ref[...]
    xn_ref[...] = xn
    m0_ref[...] = jnp.dot(xn, w_ref[...], preferred_element_type=jnp.float32)


def _tc_prep(x, g2, b2, w0):
    return pl.pallas_call(
        _prep_body,
        out_shape=[jax.ShapeDtypeStruct((N, D), jnp.float32)] * 2,
    )(x, g2, b2, w0)


_BE = 4000


def _ea_body(a_ref, w_ref, o_ref):
    o_ref[...] = lax.dot_general(
        a_ref[...], w_ref[...], (((1,), (1,)), ((), ())),
        preferred_element_type=jnp.float32)


def _tc_ea(edge_attr, w_edge):
    return pl.pallas_call(
        _ea_body,
        grid=(E2 // _BE,),
        in_specs=[
            pl.BlockSpec((_BE, D_EDGE), lambda i: (i, 0)),
            pl.BlockSpec((D, D_EDGE), lambda i: (0, 0)),
        ],
        out_specs=pl.BlockSpec((_BE, D), lambda i: (i, 0)),
        out_shape=jax.ShapeDtypeStruct((E2, D), jnp.float32),
    )(edge_attr, w_edge)


def _sc_body(m_hbm, ea_hbm, eip_hbm, zd_hbm, out_p,
             idxa, idxb, rowsa, rowsb, eab, agg_sh,
             sem_i, sem_ga, sem_gb, sem_e, sem_sa, sem_sb):
    c = lax.axis_index("c")
    s = lax.axis_index("s")
    wid = c * 16 + s
    row0 = pl.multiple_of(s * ROWS_PER_SUB, 8)

    pltpu.sync_copy(zd_hbm.at[pl.ds(row0, ROWS_PER_SUB)],
                    agg_sh.at[pl.ds(row0, ROWS_PER_SUB)])
    plsc.subcore_barrier()

    def compute(rows_ref):
        def erow(r, carry2):
            for j in range(D // 16):
                sl = pl.ds(j * 16, 16)
                v = rows_ref[r, sl] + eab[r, sl]
                rows_ref[r, sl] = jnp.maximum(v, 0.0)
            return carry2

        lax.fori_loop(0, K, erow, 0)

    def ea_copy(g):
        base = pl.multiple_of(wid * EPT + g * K, 16)
        return pltpu.async_copy(ea_hbm.at[pl.ds(base, K)], eab, sem_e)

    def pair(t, carry):
        g0 = t * 2
        g1 = g0 + 1
        ha = pltpu.async_copy(eip_hbm.at[wid, g0], idxa, sem_i)
        hb = pltpu.async_copy(eip_hbm.at[wid, g1], idxb, sem_i)
        ha.wait()
        ga = pltpu.async_copy(m_hbm.at[idxa.at[0]], rowsa, sem_ga)
        he = ea_copy(g0)
        hb.wait()
        gb = pltpu.async_copy(m_hbm.at[idxb.at[0]], rowsb, sem_gb)
        ga.wait()
        he.wait()
        compute(rowsa)
        sa = pltpu.async_copy(rowsa, agg_sh.at[idxa.at[1]], sem_sa, add=True)
        he2 = ea_copy(g1)
        gb.wait()
        he2.wait()
        compute(rowsb)
        sa.wait()
        sb = pltpu.async_copy(rowsb, agg_sh.at[idxb.at[1]], sem_sb, add=True)
        sb.wait()
        return carry

    lax.fori_loop(0, CPT // 2, pair, 0)
    hx = pltpu.async_copy(eip_hbm.at[wid, CPT - 1], idxa, sem_i)
    hx.wait()
    gx = pltpu.async_copy(m_hbm.at[idxa.at[0]], rowsa, sem_ga)
    he = ea_copy(CPT - 1)
    gx.wait()
    he.wait()
    compute(rowsa)
    pltpu.sync_copy(rowsa, agg_sh.at[idxa.at[1]], add=True)
    plsc.subcore_barrier()
    pltpu.sync_copy(agg_sh.at[pl.ds(row0, ROWS_PER_SUB)], out_p.at[c, s])


_sc_agg = functools.partial(
    pl.kernel,
    mesh=plsc.VectorSubcoreMesh(core_axis_name="c", subcore_axis_name="s"),
    out_type=jax.ShapeDtypeStruct((2, 16, ROWS_PER_SUB, D), jnp.float32),
    scratch_types=(
        [pltpu.VMEM((2, K), jnp.int32)] * 2
        + [pltpu.VMEM((K, D), jnp.float32)] * 3
        + [pltpu.VMEM_SHARED((NP, D), jnp.float32)]
        + [pltpu.SemaphoreType.DMA] * 6
    ),
)(_sc_body)


def _sc_cnt_body(dst_hbm, zd_hbm, ones_hbm, out_c, idx_d, ones_v, cnt_sh):
    c = lax.axis_index("c")
    s = lax.axis_index("s")
    wid = c * 16 + s
    row0 = pl.multiple_of(s * ROWS_PER_SUB, 8)

    pltpu.sync_copy(zd_hbm.at[pl.ds(row0, ROWS_PER_SUB)],
                    cnt_sh.at[pl.ds(row0, ROWS_PER_SUB)])
    pltpu.sync_copy(ones_hbm, ones_v)
    plsc.subcore_barrier()

    def chunk(g, carry):
        ci = wid + NW * g

        @pl.when(ci < NCHUNK)
        def _():
            base = pl.multiple_of(ci * K, K)
            pltpu.sync_copy(dst_hbm.at[pl.ds(base, K)], idx_d)
            pltpu.sync_copy(ones_v, cnt_sh.at[idx_d], add=True)

        return carry

    lax.fori_loop(0, MAX_CHUNKS, chunk, 0)
    plsc.subcore_barrier()
    pltpu.sync_copy(cnt_sh.at[pl.ds(row0, ROWS_PER_SUB)], out_c.at[c, s])


_sc_cnt = functools.partial(
    pl.kernel,
    mesh=plsc.VectorSubcoreMesh(core_axis_name="c", subcore_axis_name="s"),
    out_type=jax.ShapeDtypeStruct((2, 16, ROWS_PER_SUB, D), jnp.float32),
    scratch_types=[
        pltpu.VMEM((K,), jnp.int32),
        pltpu.VMEM((K, D), jnp.float32),
        pltpu.VMEM_SHARED((NP, D), jnp.float32),
    ],
)(_sc_cnt_body)


_BN = 1000


def _gru_body(final, p_ref, c_ref, h_ref, wih_ref, whh_ref, bih_ref, bhh_ref,
              wg_ref, h_out, *m_out):
    p = p_ref[...]
    agg_sum = p[0] + p[1]
    cp = c_ref[...]
    cnt = cp[0, :, 0:1] + cp[1, :, 0:1]
    denom = jnp.maximum(cnt, 1.0)
    agg = agg_sum / denom
    hb = h_ref[...]
    gi = lax.dot_general(agg, wih_ref[...], (((1,), (1,)), ((), ())),
                         preferred_element_type=jnp.float32) + bih_ref[...]
    gh = lax.dot_general(hb, whh_ref[...], (((1,), (1,)), ((), ())),
                         preferred_element_type=jnp.float32) + bhh_ref[...]
    r = jax.nn.sigmoid(gi[:, :D] + gh[:, :D])
    z = jax.nn.sigmoid(gi[:, D:2 * D] + gh[:, D:2 * D])
    n = jnp.tanh(gi[:, 2 * D:] + r * gh[:, 2 * D:])
    hn = (1.0 - z) * n + z * hb
    if final:
        h_out[...] = jnp.maximum(hn, 0.0)
    else:
        h_out[...] = hn
        m_out[0][...] = jnp.dot(hn, wg_ref[...],
                                preferred_element_type=jnp.float32)


def _make_gru(final):
    out_shape = [jax.ShapeDtypeStruct((N, D), jnp.float32)]
    if not final:
        out_shape.append(jax.ShapeDtypeStruct((N, D), jnp.float32))
    return pl.pallas_call(
        functools.partial(_gru_body, final),
        grid=(N // _BN,),
        in_specs=[
            pl.BlockSpec((2, _BN, D), lambda i: (0, i, 0)),
            pl.BlockSpec((2, _BN, D), lambda i: (0, i, 0)),
            pl.BlockSpec((_BN, D), lambda i: (i, 0)),
            pl.BlockSpec((3 * D, D), lambda i: (0, 0)),
            pl.BlockSpec((3 * D, D), lambda i: (0, 0)),
            pl.BlockSpec((1, 3 * D), lambda i: (0, 0)),
            pl.BlockSpec((1, 3 * D), lambda i: (0, 0)),
            pl.BlockSpec((D, D), lambda i: (0, 0)),
        ],
        out_specs=[pl.BlockSpec((_BN, D), lambda i: (i, 0))] * len(out_shape),
        out_shape=out_shape,
    )


def kernel(x, edge_index, edge_attr, bn_gamma, bn_beta, W_edge, ggc_weight,
           w_ih, w_hh, b_ih, b_hh):
    src = edge_index[0].astype(jnp.int32)
    dst = edge_index[1].astype(jnp.int32)
    srcp = jnp.pad(src.reshape(NW, EPT), ((0, 0), (0, CPT * K - EPT)),
                   constant_values=0).reshape(NW, CPT, K)
    dstp = jnp.pad(dst.reshape(NW, EPT), ((0, 0), (0, CPT * K - EPT)),
                   constant_values=N).reshape(NW, CPT, K)
    eip = jnp.stack([srcp, dstp], axis=2)
    ea_attr = jnp.pad(edge_attr, ((0, E2 - E), (0, 0)))
    g2 = bn_gamma.reshape(1, D)
    b2 = bn_beta.reshape(1, D)
    bih2 = b_ih.reshape(1, 3 * D)
    bhh2 = b_hh.reshape(1, 3 * D)
    zd = jnp.zeros((NP, D), jnp.float32)
    ones = jnp.ones((K, D), jnp.float32)

    x_norm, m0 = _tc_prep(x, g2, b2, ggc_weight[0])
    ea = _tc_ea(ea_attr, W_edge)

    c0 = _sc_cnt(dst, zd, ones)
    p0 = _sc_agg(m0, ea, eip, zd)
    p0 = p0.reshape(2, NP, D)
    c0 = c0.reshape(2, NP, D)
    h1, m1 = _make_gru(False)(p0, c0, x_norm, w_ih, w_hh, bih2, bhh2,
                              ggc_weight[1])

    p1 = _sc_agg(m1, ea, eip, zd)
    p1 = p1.reshape(2, NP, D)
    (out,) = _make_gru(True)(p1, c0, h1, w_ih, w_hh, bih2, bhh2,
                             ggc_weight[1])
    return out
